# baseline (speedup 1.0000x reference)
import jax
import jax.numpy as jnp
from jax import lax
from jax.experimental import pallas as pl
from jax.experimental.pallas import tpu as pltpu

N = 16384
V = 1000
BR = 64
NQ = 8
GRID = N // BR


def _body(x_ref, o_ref, bufs, sems):
    i = pl.program_id(0)
    k = lax.rem(i, NQ)

    @pl.when(i >= NQ)
    def _drain():
        j = i - NQ
        pltpu.make_async_copy(
            bufs.at[lax.rem(j, NQ)], o_ref.at[pl.ds(j * BR, BR), :],
            sems.at[lax.rem(j, NQ)],
        ).wait()

    x = x_ref[...]
    cols = lax.broadcasted_iota(jnp.int32, (BR, V), 1)
    bufs[k, :, :] = (cols == x).astype(jnp.float32)
    pltpu.make_async_copy(
        bufs.at[k], o_ref.at[pl.ds(i * BR, BR), :], sems.at[k]
    ).start()

    @pl.when(i == GRID - 1)
    def _final():
        for d in range(NQ):
            j = GRID - NQ + d
            pltpu.make_async_copy(
                bufs.at[lax.rem(j, NQ)], o_ref.at[pl.ds(j * BR, BR), :],
                sems.at[lax.rem(j, NQ)],
            ).wait()


def kernel(X):
    x2 = X.reshape(N, 1)
    out = pl.pallas_call(
        _body,
        grid=(GRID,),
        in_specs=[pl.BlockSpec((BR, 1), lambda i: (i, 0))],
        out_specs=pl.BlockSpec(memory_space=pl.ANY),
        out_shape=jax.ShapeDtypeStruct((N, V), jnp.float32),
        scratch_shapes=[
            pltpu.VMEM((NQ, BR, V), jnp.float32),
            pltpu.SemaphoreType.DMA((NQ,)),
        ],
    )(x2)
    return out


# speedup vs baseline: 1.9953x; 1.9953x over previous
import jax
import jax.numpy as jnp
from jax import lax
from jax.experimental import pallas as pl
from jax.experimental.pallas import tpu as pltpu

N = 16384
V = 1000
VA = 896
BR = 512
GRID = N // BR


def _body(x_ref, o_ref, bufs, sems):
    i = pl.program_id(0)
    b = lax.rem(i, 2)

    def waitpair(j):
        jb = lax.rem(j, 2)
        pltpu.make_async_copy(
            bufs.at[jb, :, pl.ds(0, VA)],
            o_ref.at[pl.ds(j * BR, BR), pl.ds(0, VA)],
            sems.at[jb, 0],
        ).wait()
        pltpu.make_async_copy(
            bufs.at[jb, :, pl.ds(VA, V - VA)],
            o_ref.at[pl.ds(j * BR, BR), pl.ds(VA, V - VA)],
            sems.at[jb, 1],
        ).wait()

    @pl.when(i >= 2)
    def _drain():
        waitpair(i - 2)

    x = x_ref[...]
    cols = lax.broadcasted_iota(jnp.int32, (BR, V), 1)
    bufs[b, :, :] = (cols == x).astype(jnp.float32)

    pltpu.make_async_copy(
        bufs.at[b, :, pl.ds(0, VA)],
        o_ref.at[pl.ds(i * BR, BR), pl.ds(0, VA)],
        sems.at[b, 0],
    ).start()
    pltpu.make_async_copy(
        bufs.at[b, :, pl.ds(VA, V - VA)],
        o_ref.at[pl.ds(i * BR, BR), pl.ds(VA, V - VA)],
        sems.at[b, 1],
    ).start()

    @pl.when(i == GRID - 1)
    def _final():
        waitpair(GRID - 2)
        waitpair(GRID - 1)


def kernel(X):
    x2 = X.reshape(N, 1)
    out = pl.pallas_call(
        _body,
        grid=(GRID,),
        in_specs=[pl.BlockSpec((BR, 1), lambda i: (i, 0))],
        out_specs=pl.BlockSpec(memory_space=pl.ANY),
        out_shape=jax.ShapeDtypeStruct((N, V), jnp.float32),
        scratch_shapes=[
            pltpu.VMEM((2, BR, V), jnp.float32),
            pltpu.SemaphoreType.DMA((2, 2)),
        ],
    )(x2)
    return out
